# Initial kernel scaffold; baseline (speedup 1.0000x reference)
import jax
import jax.numpy as jnp
from jax.experimental import pallas as pl


def kernel(x, query_tokens):
    raise NotImplementedError("write your pallas kernel here")


# speedup vs baseline: 2.1886x; 2.1886x over previous
import functools

import jax
import jax.numpy as jnp
from jax.experimental import pallas as pl
from jax.experimental.pallas import tpu as pltpu

_BN = 2048


def _pool_body(q_ref, x_ref, o_ref, acc_ref, m_ref, l_ref, *, nj):
    j = pl.program_id(1)

    @pl.when(j == 0)
    def _():
        m_ref[...] = jnp.full_like(m_ref, -1e30)
        l_ref[...] = jnp.zeros_like(l_ref)
        acc_ref[...] = jnp.zeros_like(acc_ref)

    q = q_ref[...]
    x = x_ref[...]
    s = jax.lax.dot_general(
        q, x, (((1,), (1,)), ((), ())),
        preferred_element_type=jnp.float32)

    m_prev = m_ref[:, :1]
    m_cur = jnp.max(s, axis=1, keepdims=True)
    m_new = jnp.maximum(m_prev, m_cur)
    alpha = jnp.exp(m_prev - m_new)
    p = jnp.exp(s - m_new)
    l_ref[:, :1] = l_ref[:, :1] * alpha + jnp.sum(p, axis=1, keepdims=True)
    m_ref[:, :1] = m_new
    pv = jax.lax.dot_general(
        p, x, (((1,), (0,)), ((), ())),
        preferred_element_type=jnp.float32)
    acc_ref[...] = acc_ref[...] * alpha + pv

    @pl.when(j == nj - 1)
    def _():
        o_ref[...] = acc_ref[...] / l_ref[:, :1]


def kernel(x, query_tokens):
    B, N, C = x.shape
    S = query_tokens.shape[0]
    nj = N // _BN
    q_scaled = query_tokens * (C ** -0.5)
    return pl.pallas_call(
        functools.partial(_pool_body, nj=nj),
        out_shape=jax.ShapeDtypeStruct((B, S, C), x.dtype),
        grid=(B, nj),
        in_specs=[
            pl.BlockSpec((S, C), lambda b, j: (0, 0)),
            pl.BlockSpec((None, _BN, C), lambda b, j: (b, j, 0)),
        ],
        out_specs=pl.BlockSpec((None, S, C), lambda b, j: (b, 0, 0)),
        scratch_shapes=[
            pltpu.VMEM((S, C), jnp.float32),
            pltpu.VMEM((S, 128), jnp.float32),
            pltpu.VMEM((S, 128), jnp.float32),
        ],
        compiler_params=pltpu.CompilerParams(
            dimension_semantics=("parallel", "arbitrary"),
        ),
        name="attn_pool",
    )(q_scaled, x)


# speedup vs baseline: 2.6333x; 1.2031x over previous
import functools

import jax
import jax.numpy as jnp
from jax.experimental import pallas as pl
from jax.experimental.pallas import tpu as pltpu

_BN = 4096


def _pool_body(q_ref, x_ref, o_ref, acc_ref, m_ref, l_ref, *, nj):
    j = pl.program_id(1)

    @pl.when(j == 0)
    def _():
        m_ref[...] = jnp.full_like(m_ref, -1e30)
        l_ref[...] = jnp.zeros_like(l_ref)
        acc_ref[...] = jnp.zeros_like(acc_ref)

    q = q_ref[...]
    x = x_ref[...]
    s = jax.lax.dot_general(
        q, x, (((1,), (1,)), ((), ())),
        preferred_element_type=jnp.float32)

    m_prev = m_ref[:, :1]
    m_cur = jnp.max(s, axis=1, keepdims=True)
    m_new = jnp.maximum(m_prev, m_cur)
    alpha = jnp.exp(m_prev - m_new)
    p = jnp.exp(s - m_new)
    l_ref[:, :1] = l_ref[:, :1] * alpha + jnp.sum(p, axis=1, keepdims=True)
    m_ref[:, :1] = m_new
    pv = jax.lax.dot_general(
        p, x, (((1,), (0,)), ((), ())),
        preferred_element_type=jnp.float32)
    acc_ref[...] = acc_ref[...] * alpha + pv

    @pl.when(j == nj - 1)
    def _():
        o_ref[...] = acc_ref[...] / l_ref[:, :1]


def kernel(x, query_tokens):
    B, N, C = x.shape
    S = query_tokens.shape[0]
    nj = N // _BN
    q_scaled = query_tokens * (C ** -0.5)
    return pl.pallas_call(
        functools.partial(_pool_body, nj=nj),
        out_shape=jax.ShapeDtypeStruct((B, S, C), x.dtype),
        grid=(B, nj),
        in_specs=[
            pl.BlockSpec((S, C), lambda b, j: (0, 0)),
            pl.BlockSpec((None, _BN, C), lambda b, j: (b, j, 0)),
        ],
        out_specs=pl.BlockSpec((None, S, C), lambda b, j: (b, 0, 0)),
        scratch_shapes=[
            pltpu.VMEM((S, C), jnp.float32),
            pltpu.VMEM((S, 128), jnp.float32),
            pltpu.VMEM((S, 128), jnp.float32),
        ],
        compiler_params=pltpu.CompilerParams(
            dimension_semantics=("parallel", "arbitrary"),
        ),
        name="attn_pool",
    )(q_scaled, x)


# speedup vs baseline: 2.8752x; 1.0919x over previous
import functools

import jax
import jax.numpy as jnp
from jax.experimental import pallas as pl
from jax.experimental.pallas import tpu as pltpu

_BN = 8192


def _pool_body(q_ref, x_ref, o_ref, acc_ref, m_ref, l_ref, *, nj):
    j = pl.program_id(1)

    @pl.when(j == 0)
    def _():
        m_ref[...] = jnp.full_like(m_ref, -1e30)
        l_ref[...] = jnp.zeros_like(l_ref)
        acc_ref[...] = jnp.zeros_like(acc_ref)

    q = q_ref[...]
    x = x_ref[...]
    s = jax.lax.dot_general(
        q, x, (((1,), (1,)), ((), ())),
        preferred_element_type=jnp.float32)

    m_prev = m_ref[:, :1]
    m_cur = jnp.max(s, axis=1, keepdims=True)
    m_new = jnp.maximum(m_prev, m_cur)
    alpha = jnp.exp(m_prev - m_new)
    p = jnp.exp(s - m_new)
    l_ref[:, :1] = l_ref[:, :1] * alpha + jnp.sum(p, axis=1, keepdims=True)
    m_ref[:, :1] = m_new
    pv = jax.lax.dot_general(
        p, x, (((1,), (0,)), ((), ())),
        preferred_element_type=jnp.float32)
    acc_ref[...] = acc_ref[...] * alpha + pv

    @pl.when(j == nj - 1)
    def _():
        o_ref[...] = acc_ref[...] / l_ref[:, :1]


def kernel(x, query_tokens):
    B, N, C = x.shape
    S = query_tokens.shape[0]
    nj = N // _BN
    q_scaled = query_tokens * (C ** -0.5)
    return pl.pallas_call(
        functools.partial(_pool_body, nj=nj),
        out_shape=jax.ShapeDtypeStruct((B, S, C), x.dtype),
        grid=(B, nj),
        in_specs=[
            pl.BlockSpec((S, C), lambda b, j: (0, 0)),
            pl.BlockSpec((None, _BN, C), lambda b, j: (b, j, 0)),
        ],
        out_specs=pl.BlockSpec((None, S, C), lambda b, j: (b, 0, 0)),
        scratch_shapes=[
            pltpu.VMEM((S, C), jnp.float32),
            pltpu.VMEM((S, 128), jnp.float32),
            pltpu.VMEM((S, 128), jnp.float32),
        ],
        compiler_params=pltpu.CompilerParams(
            dimension_semantics=("parallel", "arbitrary"),
            vmem_limit_bytes=56 * 1024 * 1024,
        ),
        name="attn_pool",
    )(q_scaled, x)
